# baseline (speedup 1.0000x reference)
import math

import jax
import jax.numpy as jnp
import numpy as np
from jax import lax
from jax.experimental import pallas as pl
from jax.experimental.pallas import tpu as pltpu
from jax.experimental.pallas import tpu_sc as plsc

EMBED_DIM = 1024
MAX_LENGTH = 2048
B, L = 4, 2048

NUM_WORKERS = 32
SEQ_PER_W = L // NUM_WORKERS
CHUNK = 8
LANES = 16
NSLICE = EMBED_DIM // LANES
CHUNKS_PER_W = (B * SEQ_PER_W) // CHUNK


def _make_pos_enc_np():
    position = np.arange(MAX_LENGTH)[:, None].astype(np.float32)
    div_term = np.exp(
        np.arange(0, EMBED_DIM, 2).astype(np.float32)
        * (-math.log(10000.0) / EMBED_DIM)
    )
    pe = np.zeros((MAX_LENGTH, EMBED_DIM), dtype=np.float32)
    pe[:, 0::2] = np.sin(position * div_term)
    pe[:, 1::2] = np.cos(position * div_term)
    return pe


_POS_ENC = _make_pos_enc_np()[:L]


def _sc_body(x_hbm, lvl_hbm, pos_hbm, table_hbm, out_hbm,
             pos_v, row_v, lvl_v, xbuf, row_sem):
    cid = lax.axis_index("c")
    sid = lax.axis_index("s")
    w = cid * 16 + sid
    seq0 = w * SEQ_PER_W

    pltpu.sync_copy(pos_hbm.at[pl.ds(seq0, SEQ_PER_W)], pos_v)
    pltpu.sync_copy(lvl_hbm, lvl_v)
    pltpu.async_copy(table_hbm.at[lvl_v], row_v, row_sem).wait()

    def fold_row(r, _):
        for s in range(NSLICE):
            sl = pl.ds(s * LANES, LANES)
            pos_v[r, sl] = pos_v[r, sl] + row_v[0, sl]
        return 0

    lax.fori_loop(0, SEQ_PER_W, fold_row, 0)

    def do_chunk(i, _):
        b = i // (SEQ_PER_W // CHUNK)
        c = i % (SEQ_PER_W // CHUNK)
        g0 = b * L + seq0 + c * CHUNK
        pltpu.sync_copy(x_hbm.at[pl.ds(g0, CHUNK)], xbuf)

        def add_row(r, _):
            for s in range(NSLICE):
                sl = pl.ds(s * LANES, LANES)
                xbuf[r, sl] = xbuf[r, sl] + pos_v[c * CHUNK + r, sl]
            return 0

        lax.fori_loop(0, CHUNK, add_row, 0)
        pltpu.sync_copy(xbuf, out_hbm.at[pl.ds(g0, CHUNK)])
        return 0

    lax.fori_loop(0, CHUNKS_PER_W, do_chunk, 0)


def kernel(x, level, summary_table):
    x2d = x.reshape(B * L, EMBED_DIM)
    lvl_arr = jnp.reshape(jnp.asarray(level, jnp.int32), (1,))
    pos_enc = jnp.asarray(_POS_ENC)

    mesh = plsc.VectorSubcoreMesh(core_axis_name="c", subcore_axis_name="s")
    fn = pl.kernel(
        _sc_body,
        out_type=jax.ShapeDtypeStruct((B * L, EMBED_DIM), jnp.float32),
        mesh=mesh,
        scratch_types=[
            pltpu.VMEM((SEQ_PER_W, EMBED_DIM), jnp.float32),
            pltpu.VMEM((1, EMBED_DIM), jnp.float32),
            pltpu.VMEM((1,), jnp.int32),
            pltpu.VMEM((CHUNK, EMBED_DIM), jnp.float32),
            pltpu.SemaphoreType.DMA,
        ],
    )
    out2d = fn(x2d, lvl_arr, pos_enc, summary_table)
    return out2d.reshape(B, L, EMBED_DIM)


# speedup vs baseline: 1.2538x; 1.2538x over previous
import math

import jax
import jax.numpy as jnp
import numpy as np
from jax import lax
from jax.experimental import pallas as pl
from jax.experimental.pallas import tpu as pltpu
from jax.experimental.pallas import tpu_sc as plsc

EMBED_DIM = 1024
MAX_LENGTH = 2048
B, L = 4, 2048

NUM_WORKERS = 32
SEQ_PER_W = L // NUM_WORKERS
CHUNK = 8
LANES = 16
NSLICE = EMBED_DIM // LANES
CHUNKS_PER_W = (B * SEQ_PER_W) // CHUNK


def _make_pos_enc_np():
    position = np.arange(MAX_LENGTH)[:, None].astype(np.float32)
    div_term = np.exp(
        np.arange(0, EMBED_DIM, 2).astype(np.float32)
        * (-math.log(10000.0) / EMBED_DIM)
    )
    pe = np.zeros((MAX_LENGTH, EMBED_DIM), dtype=np.float32)
    pe[:, 0::2] = np.sin(position * div_term)
    pe[:, 1::2] = np.cos(position * div_term)
    return pe


_POS_ENC = _make_pos_enc_np()[:L]


def _sc_body(x_hbm, lvl_hbm, pos_hbm, table_hbm, out_hbm,
             pos_v, row_v, lvl_v, ibuf, obuf, row_sem, in_sems, out_sems):
    cid = lax.axis_index("c")
    sid = lax.axis_index("s")
    w = cid * 16 + sid
    seq0 = w * SEQ_PER_W

    def chunk_base(i):
        cpb = SEQ_PER_W // CHUNK
        return (i // cpb) * L + seq0 + (i % cpb) * CHUNK

    def in_copy(i, p):
        return pltpu.make_async_copy(
            x_hbm.at[pl.ds(chunk_base(i), CHUNK)], ibuf.at[p], in_sems.at[p])

    def out_copy(i, p):
        return pltpu.make_async_copy(
            obuf.at[p], out_hbm.at[pl.ds(chunk_base(i), CHUNK)], out_sems.at[p])

    in_copy(0, 0).start()
    in_copy(1, 1).start()

    pltpu.sync_copy(pos_hbm.at[pl.ds(seq0, SEQ_PER_W)], pos_v)
    pltpu.sync_copy(lvl_hbm, lvl_v)
    pltpu.async_copy(table_hbm.at[lvl_v], row_v, row_sem).wait()

    def fold_row(r, _):
        for s in range(NSLICE):
            sl = pl.ds(s * LANES, LANES)
            pos_v[r, sl] = pos_v[r, sl] + row_v[0, sl]
        return 0

    lax.fori_loop(0, SEQ_PER_W, fold_row, 0)

    def do_chunk(i, _):
        p = lax.rem(i, 2)
        c = lax.rem(i, SEQ_PER_W // CHUNK)
        in_copy(i, p).wait()

        @pl.when(i >= 2)
        def _():
            out_copy(i - 2, p).wait()

        def add_row(r, _):
            for s in range(NSLICE):
                sl = pl.ds(s * LANES, LANES)
                obuf[p, r, sl] = ibuf[p, r, sl] + pos_v[c * CHUNK + r, sl]
            return 0

        lax.fori_loop(0, CHUNK, add_row, 0)
        out_copy(i, p).start()

        @pl.when(i < CHUNKS_PER_W - 2)
        def _():
            in_copy(i + 2, p).start()

        return 0

    lax.fori_loop(0, CHUNKS_PER_W, do_chunk, 0)
    out_copy(CHUNKS_PER_W - 2, 0).wait()
    out_copy(CHUNKS_PER_W - 1, 1).wait()


def kernel(x, level, summary_table):
    x2d = x.reshape(B * L, EMBED_DIM)
    lvl_arr = jnp.reshape(jnp.asarray(level, jnp.int32), (1,))
    pos_enc = jnp.asarray(_POS_ENC)

    mesh = plsc.VectorSubcoreMesh(core_axis_name="c", subcore_axis_name="s")
    fn = pl.kernel(
        _sc_body,
        out_type=jax.ShapeDtypeStruct((B * L, EMBED_DIM), jnp.float32),
        mesh=mesh,
        scratch_types=[
            pltpu.VMEM((SEQ_PER_W, EMBED_DIM), jnp.float32),
            pltpu.VMEM((1, EMBED_DIM), jnp.float32),
            pltpu.VMEM((1,), jnp.int32),
            pltpu.VMEM((2, CHUNK, EMBED_DIM), jnp.float32),
            pltpu.VMEM((2, CHUNK, EMBED_DIM), jnp.float32),
            pltpu.SemaphoreType.DMA,
            pltpu.SemaphoreType.DMA((2,)),
            pltpu.SemaphoreType.DMA((2,)),
        ],
    )
    out2d = fn(x2d, lvl_arr, pos_enc, summary_table)
    return out2d.reshape(B, L, EMBED_DIM)
